# baseline (speedup 1.0000x reference)
import functools

import jax
import jax.numpy as jnp
from jax import lax
from jax.experimental import pallas as pl
from jax.experimental.pallas import tpu as pltpu
from jax.experimental.pallas import tpu_sc as plsc

NC, NS, L = 2, 16, 16
NW = NC * NS
B, T, D = 256, 300, 512
BB = 32
NBB = B // BB
UNITS = T * NBB
NB = 5
JPW = UNITS // NW

_mesh = plsc.VectorSubcoreMesh(core_axis_name="c", subcore_axis_name="s")


@functools.partial(
    pl.kernel,
    out_type=jax.ShapeDtypeStruct((T, B, D), jnp.float32),
    mesh=_mesh,
    scratch_types=[
        pltpu.VMEM((NB, BB, D), jnp.float32),
        pltpu.VMEM((NB, 1, D), jnp.float32),
        pltpu.VMEM((NB, 1, BB), jnp.int32),
        pltpu.SemaphoreType.DMA((NB,)),
        pltpu.SemaphoreType.DMA((NB,)),
        pltpu.SemaphoreType.DMA((NB,)),
        pltpu.SemaphoreType.DMA((NB,)),
    ],
)
def _embed(idxf_hbm, tok_hbm, posf_hbm, out_hbm,
           rows_v, pos_v, idx_v, gsem, psem, isem, ssem):
    wid = lax.axis_index("s") * NC + lax.axis_index("c")

    def unit(j):
        u = wid + j * NW
        return u // NBB, BB * lax.rem(u, NBB)

    def make_idxcopy(j, b):
        t, b0 = unit(j)
        off = pl.multiple_of(t * B + b0, 8)
        return pltpu.make_async_copy(
            idxf_hbm.at[pl.ds(off, BB)], idx_v.at[b, 0], isem.at[b])

    def make_poscopy(j, b):
        t, _ = unit(j)
        off = pl.multiple_of(t * D, 8)
        return pltpu.make_async_copy(
            posf_hbm.at[pl.ds(off, D)], pos_v.at[b, 0], psem.at[b])

    def make_gather(b):
        return pltpu.make_async_copy(
            tok_hbm.at[idx_v.at[b, 0]], rows_v.at[b], gsem.at[b])

    def make_store(j, b):
        t, b0 = unit(j)
        return pltpu.make_async_copy(
            rows_v.at[b],
            out_hbm.at[t, pl.ds(pl.multiple_of(b0, 8), BB), :], ssem.at[b])

    for j0 in range(NB):
        make_idxcopy(j0, j0).start()
        make_poscopy(j0, j0).start()
    for j0 in range(3):
        make_idxcopy(j0, j0).wait()
        make_gather(j0).start()

    def outer(g, _):
        for s in range(NB):
            j = g * NB + s
            b = s

            make_gather(b).wait()
            make_poscopy(j, b).wait()

            for c in range(D // L):
                sl = pl.ds(c * L, L)
                pc = pos_v[b, 0, sl]

                def add4(r4, _, sl=sl, pc=pc, b=b):
                    r = r4 * 4
                    for i in range(4):
                        rows_v[b, r + i, sl] = rows_v[b, r + i, sl] + pc
                    return 0

                lax.fori_loop(0, BB // 4, add4, 0)

            @pl.when(j >= 1)
            def _():
                make_store(j - 1, (s - 1) % NB).wait()

            make_store(j, b).start()

            @pl.when(j + NB < JPW)
            def _():
                make_idxcopy(j + NB, b).start()
                make_poscopy(j + NB, b).start()

            @pl.when(j + 3 < JPW)
            def _():
                make_idxcopy(j + 3, (s + 3) % NB).wait()
                make_gather((s + 3) % NB).start()
        return 0

    lax.fori_loop(0, JPW // NB, outer, 0)
    make_store(JPW - 1, (JPW - 1) % NB).wait()


def kernel(idx, token_table, pos_table):
    idxf = idx.T.reshape(T * B)
    posf = pos_table.reshape(T * D)
    out = _embed(idxf, token_table, posf)
    return out.transpose(1, 0, 2)


# speedup vs baseline: 1.0387x; 1.0387x over previous
import functools

import jax
import jax.numpy as jnp
from jax import lax
from jax.experimental import pallas as pl
from jax.experimental.pallas import tpu as pltpu
from jax.experimental.pallas import tpu_sc as plsc

NC, NS, L = 2, 16, 16
NW = NC * NS
B, T, D = 256, 300, 512
BB = 64
NBB = B // BB
UNITS = T * NBB
NB = 3
JMAX = 39

_mesh = plsc.VectorSubcoreMesh(core_axis_name="c", subcore_axis_name="s")


@functools.partial(
    pl.kernel,
    out_type=jax.ShapeDtypeStruct((T, B, D), jnp.float32),
    mesh=_mesh,
    scratch_types=[
        pltpu.VMEM((NB, BB, D), jnp.float32),
        pltpu.VMEM((NB, 1, D), jnp.float32),
        pltpu.VMEM((NB, 1, BB), jnp.int32),
        pltpu.SemaphoreType.DMA((NB,)),
        pltpu.SemaphoreType.DMA((NB,)),
        pltpu.SemaphoreType.DMA((NB,)),
        pltpu.SemaphoreType.DMA((NB,)),
    ],
)
def _embed(idxf_hbm, tok_hbm, posf_hbm, out_hbm,
           rows_v, pos_v, idx_v, gsem, psem, isem, ssem):
    wid = lax.axis_index("s") * NC + lax.axis_index("c")

    def unit(j):
        u = wid + j * NW
        return u // NBB, BB * lax.rem(u, NBB)

    def valid(j):
        return wid + j * NW < UNITS

    def make_idxcopy(j, b):
        t, b0 = unit(j)
        off = pl.multiple_of(t * B + b0, 8)
        return pltpu.make_async_copy(
            idxf_hbm.at[pl.ds(off, BB)], idx_v.at[b, 0], isem.at[b])

    def make_poscopy(j, b):
        t, _ = unit(j)
        off = pl.multiple_of(t * D, 8)
        return pltpu.make_async_copy(
            posf_hbm.at[pl.ds(off, D)], pos_v.at[b, 0], psem.at[b])

    def make_gather(b):
        return pltpu.make_async_copy(
            tok_hbm.at[idx_v.at[b, 0]], rows_v.at[b], gsem.at[b])

    def make_store(j, b):
        t, b0 = unit(j)
        return pltpu.make_async_copy(
            rows_v.at[b],
            out_hbm.at[t, pl.ds(pl.multiple_of(b0, 8), BB), :], ssem.at[b])

    for j0 in range(NB):
        make_idxcopy(j0, j0).start()
        make_poscopy(j0, j0).start()
    for j0 in range(2):
        make_idxcopy(j0, j0).wait()
        make_gather(j0).start()

    def outer(g, _):
        for s in range(NB):
            j = g * NB + s
            b = s

            @pl.when(valid(j))
            def _():
                make_gather(b).wait()
                make_poscopy(j, b).wait()

                for c in range(D // L):
                    sl = pl.ds(c * L, L)
                    pc = pos_v[b, 0, sl]

                    def add4(r4, _, sl=sl, pc=pc, b=b):
                        r = r4 * 4
                        for i in range(4):
                            rows_v[b, r + i, sl] = rows_v[b, r + i, sl] + pc
                        return 0

                    lax.fori_loop(0, BB // 4, add4, 0)

            @pl.when((j >= 1) & valid(j - 1))
            def _():
                make_store(j - 1, (s - 1) % NB).wait()

            @pl.when(valid(j))
            def _():
                make_store(j, b).start()

            @pl.when(valid(j + 3))
            def _():
                make_idxcopy(j + 3, b).start()
                make_poscopy(j + 3, b).start()

            @pl.when(valid(j + 2))
            def _():
                make_idxcopy(j + 2, (s + 2) % NB).wait()
                make_gather((s + 2) % NB).start()
        return 0

    lax.fori_loop(0, JMAX // NB, outer, 0)


def kernel(idx, token_table, pos_table):
    idxf = idx.T.reshape(T * B)
    posf = pos_table.reshape(T * D)
    out = _embed(idxf, token_table, posf)
    return out.transpose(1, 0, 2)
